# baseline (speedup 1.0000x reference)
import functools

import jax
import jax.numpy as jnp
from jax import lax
from jax.experimental import pallas as pl
from jax.experimental.pallas import tpu as pltpu
from jax.experimental.pallas import tpu_sc as plsc

C = 26
V = 100000
D = 32
B = 16384
W = 16
OUT = C * D + W
EPS = 1e-5

_CB = 128
_INFO = plsc.get_sparse_core_info()
_NC, _NS, _L = _INFO.num_cores, _INFO.num_subcores, _INFO.num_lanes
_NW = _NC * _NS
_BPW = B // _NW
_NCHUNK = _BPW // _CB


def _rsqrt(x):
    xi = plsc.bitcast(x, jnp.int32)
    yi = jnp.int32(0x5F3759DF) - lax.shift_right_logical(xi, 1)
    y = plsc.bitcast(yi, jnp.float32)
    for _ in range(3):
        y = y * (1.5 - 0.5 * x * y * y)
    return y


def _body(deep_hbm, wide_hbm, tabrows_hbm, lnw_hbm, lnb_hbm, out_hbm,
          idx_v, jb_v, wide_v, land_v, trans_v, wout_v, lnwb_v, sem):
    wid = lax.axis_index("s") * _NC + lax.axis_index("c")
    base0 = wid * _BPW
    iota = lax.broadcasted_iota(jnp.int32, (_L,), 0)

    pltpu.sync_copy(lnw_hbm, lnwb_v.at[0])
    pltpu.sync_copy(lnb_hbm, lnwb_v.at[1])

    pltpu.sync_copy(deep_hbm.at[:, pl.ds(base0, _BPW)], idx_v)
    pltpu.sync_copy(wide_hbm.at[:, pl.ds(base0, _BPW)], wide_v)

    for c in range(C):
        def cvt(k, _, c=c):
            v = idx_v[c, pl.ds(k * _L, _L)]
            g = lax.shift_right_logical(c * V + v, 1 + 1)
            idx_v[c, pl.ds(k * _L, _L)] = g
            jb_v[c, pl.ds(k * _L, _L)] = lax.shift_left(v & 3, 5)
            return 0
        lax.fori_loop(0, _BPW // _L, cvt, 0)

    def chunk(k, _):
        base = base0 + k * _CB

        for c in range(C):
            cp = pltpu.async_copy(
                tabrows_hbm.at[idx_v.at[c, pl.ds(k * _CB, _CB)]], land_v, sem)
            cp.wait()

            def grp(k2, _, c=c):
                jb = jb_v[c, pl.ds(k * _CB + k2 * _L, _L)]
                rows = k2 * _L + iota

                def dim(d, _):
                    y = plsc.load_gather(land_v, [rows, jb + d])
                    plsc.store_scatter(
                        trans_v, [jnp.full((_L,), d, jnp.int32),
                                  k2 * _L + iota], y)
                    return 0
                lax.fori_loop(0, D, dim, 0)
                return 0
            lax.fori_loop(0, _CB // _L, grp, 0)
            pltpu.sync_copy(trans_v,
                            out_hbm.at[pl.ds(c * D, D), pl.ds(base, _CB)])

        def wgrp(k2, _):
            xs = [wide_v[f, pl.ds(k * _CB + k2 * _L, _L)] for f in range(W)]
            s = xs[0]
            for f in range(1, W):
                s = s + xs[f]
            mean = s * (1.0 / W)
            var = (xs[0] - mean) * (xs[0] - mean)
            for f in range(1, W):
                var = var + (xs[f] - mean) * (xs[f] - mean)
            r = _rsqrt(var * (1.0 / W) + EPS)
            for f in range(W):
                lw = plsc.load_gather(
                    lnwb_v, [jnp.full((_L,), 0, jnp.int32),
                             jnp.full((_L,), f, jnp.int32)])
                lb = plsc.load_gather(
                    lnwb_v, [jnp.full((_L,), 1, jnp.int32),
                             jnp.full((_L,), f, jnp.int32)])
                wout_v[f, pl.ds(k2 * _L, _L)] = (xs[f] - mean) * r * lw + lb
            return 0
        lax.fori_loop(0, _CB // _L, wgrp, 0)
        pltpu.sync_copy(wout_v, out_hbm.at[pl.ds(C * D, W), pl.ds(base, _CB)])
        return 0

    lax.fori_loop(0, _NCHUNK, chunk, 0)


def kernel(deep_in, wide_in, tables, ln_w, ln_b):
    tabrows = jnp.reshape(tables, (C * V * D // 128, 128))
    mesh = plsc.VectorSubcoreMesh(core_axis_name="c", subcore_axis_name="s")
    k = functools.partial(
        pl.kernel,
        mesh=mesh,
        compiler_params=pltpu.CompilerParams(needs_layout_passes=False),
        out_type=jax.ShapeDtypeStruct((OUT, B), jnp.float32),
        scratch_types=[
            pltpu.VMEM((C, _BPW), jnp.int32),
            pltpu.VMEM((C, _BPW), jnp.int32),
            pltpu.VMEM((W, _BPW), jnp.float32),
            pltpu.VMEM((_CB, 128), jnp.float32),
            pltpu.VMEM((D, _CB), jnp.float32),
            pltpu.VMEM((W, _CB), jnp.float32),
            pltpu.VMEM((2, W), jnp.float32),
            pltpu.SemaphoreType.DMA,
        ],
    )(_body)
    out_t = k(deep_in, wide_in, tabrows, ln_w, ln_b)
    return jnp.transpose(out_t)
